# Initial kernel scaffold; baseline (speedup 1.0000x reference)
import jax
import jax.numpy as jnp
from jax.experimental import pallas as pl


def kernel(center_words, target_words, all_vocabs, embedding_v, embedding_u):
    raise NotImplementedError("write your pallas kernel here")


# speedup vs baseline: 77.0704x; 77.0704x over previous
import functools

import jax
import jax.numpy as jnp
from jax import lax
from jax.experimental import pallas as pl
from jax.experimental.pallas import tpu as pltpu
from jax.experimental.pallas import tpu_sc as plsc

_VOCAB = 1000
_VPAD = 1024
_EMBED = 128
_BATCH = 1024
_NC = 2
_NS = 16
_NW = _NC * _NS
_BPW = _BATCH // _NW
_L = 16
_NFULL = _VOCAB // _L
_TAIL_START = _VOCAB - _L
_TAIL_KEEP = _NFULL * _L - _TAIL_START


def _mm_exp_body(v_ref, ut_ref, em_ref):
    m = jnp.dot(v_ref[...], ut_ref[...], preferred_element_type=jnp.float32)
    em_ref[...] = jnp.exp(m)


def _mm_exp(v, ut_pad):
    return pl.pallas_call(
        _mm_exp_body,
        out_shape=jax.ShapeDtypeStruct((_VOCAB, _VPAD), jnp.float32),
    )(v, ut_pad)


_sc_mesh = plsc.VectorSubcoreMesh(core_axis_name="c", subcore_axis_name="s")


@functools.partial(
    pl.kernel,
    mesh=_sc_mesh,
    compiler_params=pltpu.CompilerParams(
        use_tc_tiling_on_sc=False, needs_layout_passes=False),
    out_type=(
        jax.ShapeDtypeStruct((_BATCH,), jnp.float32),
        jax.ShapeDtypeStruct((_BATCH,), jnp.float32),
    ),
    scratch_types=[
        pltpu.VMEM((_BPW,), jnp.int32),
        pltpu.VMEM((_BPW,), jnp.int32),
        pltpu.VMEM((_BPW * _VOCAB,), jnp.int32),
        pltpu.VMEM((_BPW, _VPAD), jnp.float32),
        pltpu.VMEM((_BPW,), jnp.float32),
        pltpu.VMEM((_BPW,), jnp.float32),
        pltpu.SemaphoreType.DMA,
    ],
)
def _sc_gather(em_hbm, c_hbm, t_hbm, a_hbm, denom_hbm, escore_hbm,
               cidx, tidx, av, rows, dstage, estage, sem):
    wid = lax.axis_index("s") * _NC + lax.axis_index("c")
    base = wid * _BPW
    pltpu.sync_copy(c_hbm.at[pl.ds(base, _BPW)], cidx)
    pltpu.sync_copy(t_hbm.at[pl.ds(base, _BPW)], tidx)
    pltpu.sync_copy(a_hbm.at[pl.ds(base * _VOCAB, _BPW * _VOCAB)], av)
    pltpu.async_copy(em_hbm.at[cidx], rows, sem).wait()

    lanes = lax.iota(jnp.int32, _L)

    def _row(i, dsums):
        row = jnp.full((_L,), 0, jnp.int32) + i
        abase = i * _VOCAB
        acc = jnp.zeros((_L,), jnp.float32)
        for j in range(_NFULL):
            col = av[pl.ds(abase + j * _L, _L)]
            acc = acc + plsc.load_gather(rows, [row, col])
        col = av[pl.ds(abase + _TAIL_START, _L)]
        tail = plsc.load_gather(rows, [row, col])
        acc = acc + jnp.where(lanes >= _TAIL_KEEP, tail, 0.0)
        s = jnp.sum(acc)
        return jnp.where(lanes == (i % _L), s, dsums)

    for g in range(_BPW // _L):
        dsums = lax.fori_loop(g * _L, (g + 1) * _L, _row,
                              jnp.zeros((_L,), jnp.float32))
        dstage[pl.ds(g * _L, _L)] = dsums
        ridx = lanes + g * _L
        tcol = tidx[pl.ds(g * _L, _L)]
        estage[pl.ds(g * _L, _L)] = plsc.load_gather(rows, [ridx, tcol])

    pltpu.sync_copy(dstage, denom_hbm.at[pl.ds(base, _BPW)])
    pltpu.sync_copy(estage, escore_hbm.at[pl.ds(base, _BPW)])


def _nll_body(d_ref, e_ref, o_ref):
    t = jnp.sum(jnp.log(d_ref[...])) - jnp.sum(jnp.log(e_ref[...]))
    o_ref[0, 0] = t * (1.0 / _BATCH)


def _nll(denom, escore):
    return pl.pallas_call(
        _nll_body,
        out_shape=jax.ShapeDtypeStruct((1, 1), jnp.float32),
        out_specs=pl.BlockSpec(memory_space=pltpu.SMEM),
    )(denom.reshape(8, 128), escore.reshape(8, 128))


def kernel(center_words, target_words, all_vocabs, embedding_v, embedding_u):
    c32 = center_words.reshape(-1).astype(jnp.int32)
    t32 = target_words.reshape(-1).astype(jnp.int32)
    a32 = all_vocabs.astype(jnp.int32).reshape(-1)
    ut_pad = jnp.pad(embedding_u, ((0, _VPAD - _VOCAB), (0, 0))).T
    em = _mm_exp(embedding_v, ut_pad)
    denom, escore = _sc_gather(em, c32, t32, a32)
    return _nll(denom, escore)[0, 0]


# speedup vs baseline: 79.1710x; 1.0273x over previous
import functools

import jax
import jax.numpy as jnp
from jax import lax
from jax.experimental import pallas as pl
from jax.experimental.pallas import tpu as pltpu
from jax.experimental.pallas import tpu_sc as plsc

_VOCAB = 1000
_VPAD = 1024
_EMBED = 128
_BATCH = 1024
_NC = 2
_NS = 16
_NW = _NC * _NS
_BPW = _BATCH // _NW
_L = 16
_NFULL = _VOCAB // _L
_TAIL_START = _VOCAB - _L
_TAIL_KEEP = _NFULL * _L - _TAIL_START


def _mm_exp_body(v_ref, ut_ref, em_ref):
    m = jnp.dot(v_ref[...], ut_ref[...], preferred_element_type=jnp.float32)
    em_ref[...] = jnp.exp(m)


def _mm_exp(v, ut_pad):
    return pl.pallas_call(
        _mm_exp_body,
        out_shape=jax.ShapeDtypeStruct((_VOCAB, _VPAD), jnp.float32),
    )(v, ut_pad)


_sc_mesh = plsc.VectorSubcoreMesh(core_axis_name="c", subcore_axis_name="s")


@functools.partial(
    pl.kernel,
    mesh=_sc_mesh,
    compiler_params=pltpu.CompilerParams(
        use_tc_tiling_on_sc=False, needs_layout_passes=False,
        disable_bounds_checks=True),
    out_type=(
        jax.ShapeDtypeStruct((_BATCH,), jnp.float32),
        jax.ShapeDtypeStruct((_BATCH,), jnp.float32),
    ),
    scratch_types=[
        pltpu.VMEM((_BPW,), jnp.int32),
        pltpu.VMEM((_BPW,), jnp.int32),
        pltpu.VMEM((_BPW * _VOCAB,), jnp.int32),
        pltpu.VMEM((_BPW, _VPAD), jnp.float32),
        pltpu.VMEM((_BPW,), jnp.float32),
        pltpu.VMEM((_BPW,), jnp.float32),
        pltpu.SemaphoreType.DMA,
        pltpu.SemaphoreType.DMA,
    ],
)
def _sc_gather(em_hbm, c_hbm, t_hbm, a_hbm, denom_hbm, escore_hbm,
               cidx, tidx, av, rows, dstage, estage, sem, sem2):
    wid = lax.axis_index("s") * _NC + lax.axis_index("c")
    base = wid * _BPW
    pltpu.sync_copy(c_hbm.at[pl.ds(base, _BPW)], cidx)
    av_cp = pltpu.async_copy(
        a_hbm.at[pl.ds(base * _VOCAB, _BPW * _VOCAB)], av, sem2)
    rows_cp = pltpu.async_copy(em_hbm.at[cidx], rows, sem)
    pltpu.sync_copy(t_hbm.at[pl.ds(base, _BPW)], tidx)
    av_cp.wait()
    rows_cp.wait()

    lanes = lax.iota(jnp.int32, _L)

    def _row(i, dsums):
        row = jnp.full((_L,), 0, jnp.int32) + i
        abase = i * _VOCAB
        acc = jnp.zeros((_L,), jnp.float32)
        for j in range(_NFULL):
            col = av[pl.ds(abase + j * _L, _L)]
            acc = acc + plsc.load_gather(rows, [row, col])
        col = av[pl.ds(abase + _TAIL_START, _L)]
        tail = plsc.load_gather(rows, [row, col])
        acc = acc + jnp.where(lanes >= _TAIL_KEEP, tail, 0.0)
        s = jnp.sum(acc)
        return jnp.where(lanes == (i % _L), s, dsums)

    for g in range(_BPW // _L):
        dsums = lax.fori_loop(g * _L, (g + 1) * _L, _row,
                              jnp.zeros((_L,), jnp.float32))
        dstage[pl.ds(g * _L, _L)] = dsums
        ridx = lanes + g * _L
        tcol = tidx[pl.ds(g * _L, _L)]
        estage[pl.ds(g * _L, _L)] = plsc.load_gather(rows, [ridx, tcol])

    pltpu.sync_copy(dstage, denom_hbm.at[pl.ds(base, _BPW)])
    pltpu.sync_copy(estage, escore_hbm.at[pl.ds(base, _BPW)])


def _nll_body(d_ref, e_ref, o_ref):
    t = jnp.sum(jnp.log(d_ref[...])) - jnp.sum(jnp.log(e_ref[...]))
    o_ref[0, 0] = t * (1.0 / _BATCH)


def _nll(denom, escore):
    return pl.pallas_call(
        _nll_body,
        out_shape=jax.ShapeDtypeStruct((1, 1), jnp.float32),
        out_specs=pl.BlockSpec(memory_space=pltpu.SMEM),
    )(denom.reshape(8, 128), escore.reshape(8, 128))


def kernel(center_words, target_words, all_vocabs, embedding_v, embedding_u):
    c32 = center_words.reshape(-1).astype(jnp.int32)
    t32 = target_words.reshape(-1).astype(jnp.int32)
    a32 = all_vocabs.astype(jnp.int32).reshape(-1)
    ut_pad = jnp.pad(embedding_u, ((0, _VPAD - _VOCAB), (0, 0))).T
    em = _mm_exp(embedding_v, ut_pad)
    denom, escore = _sc_gather(em, c32, t32, a32)
    return _nll(denom, escore)[0, 0]


# speedup vs baseline: 82.5790x; 1.0430x over previous
import functools

import jax
import jax.numpy as jnp
from jax import lax
from jax.experimental import pallas as pl
from jax.experimental.pallas import tpu as pltpu
from jax.experimental.pallas import tpu_sc as plsc

_VOCAB = 1000
_VPAD = 1024
_EMBED = 128
_BATCH = 1024
_NC = 2
_NS = 16
_NW = _NC * _NS
_BPW = _BATCH // _NW
_L = 16
_NFULL = _VOCAB // _L
_TAIL_START = _VOCAB - _L
_TAIL_KEEP = _NFULL * _L - _TAIL_START


def _mm_exp_body(v_ref, u_ref, em_ref):
    m = lax.dot_general(v_ref[...], u_ref[...],
                        dimension_numbers=(((1,), (1,)), ((), ())),
                        preferred_element_type=jnp.float32)
    em_ref[:, pl.ds(0, _VOCAB)] = jnp.exp(m)


def _mm_exp(v, u):
    return pl.pallas_call(
        _mm_exp_body,
        out_shape=jax.ShapeDtypeStruct((_VOCAB, _VPAD), jnp.float32),
    )(v, u)


_sc_mesh = plsc.VectorSubcoreMesh(core_axis_name="c", subcore_axis_name="s")


@functools.partial(
    pl.kernel,
    mesh=_sc_mesh,
    compiler_params=pltpu.CompilerParams(
        use_tc_tiling_on_sc=False, needs_layout_passes=False,
        disable_bounds_checks=True),
    out_type=(
        jax.ShapeDtypeStruct((_BATCH,), jnp.float32),
        jax.ShapeDtypeStruct((_BATCH,), jnp.float32),
    ),
    scratch_types=[
        pltpu.VMEM((_BPW,), jnp.int32),
        pltpu.VMEM((_BPW,), jnp.int32),
        pltpu.VMEM((_BPW * _VOCAB,), jnp.int32),
        pltpu.VMEM((_BPW, _VPAD), jnp.float32),
        pltpu.VMEM((_BPW,), jnp.float32),
        pltpu.VMEM((_BPW,), jnp.float32),
        pltpu.SemaphoreType.DMA,
        pltpu.SemaphoreType.DMA,
    ],
)
def _sc_gather(em_hbm, c_hbm, t_hbm, a_hbm, denom_hbm, escore_hbm,
               cidx, tidx, av, rows, dstage, estage, sem, sem2):
    wid = lax.axis_index("s") * _NC + lax.axis_index("c")
    base = wid * _BPW
    pltpu.sync_copy(c_hbm.at[pl.ds(base, _BPW)], cidx)
    av_cp = pltpu.async_copy(
        a_hbm.at[pl.ds(base * _VOCAB, _BPW * _VOCAB)], av, sem2)
    rows_cp = pltpu.async_copy(em_hbm.at[cidx], rows, sem)
    pltpu.sync_copy(t_hbm.at[pl.ds(base, _BPW)], tidx)
    av_cp.wait()
    rows_cp.wait()

    lanes = lax.iota(jnp.int32, _L)

    def _row(i, dsums):
        row = jnp.full((_L,), 0, jnp.int32) + i
        abase = i * _VOCAB
        acc = jnp.zeros((_L,), jnp.float32)
        for j in range(_NFULL):
            col = av[pl.ds(abase + j * _L, _L)]
            acc = acc + plsc.load_gather(rows, [row, col])
        col = av[pl.ds(abase + _TAIL_START, _L)]
        tail = plsc.load_gather(rows, [row, col])
        acc = acc + jnp.where(lanes >= _TAIL_KEEP, tail, 0.0)
        s = jnp.sum(acc)
        return jnp.where(lanes == (i % _L), s, dsums)

    for g in range(_BPW // _L):
        dsums = lax.fori_loop(g * _L, (g + 1) * _L, _row,
                              jnp.zeros((_L,), jnp.float32))
        dstage[pl.ds(g * _L, _L)] = dsums
        ridx = lanes + g * _L
        tcol = tidx[pl.ds(g * _L, _L)]
        estage[pl.ds(g * _L, _L)] = plsc.load_gather(rows, [ridx, tcol])

    pltpu.sync_copy(dstage, denom_hbm.at[pl.ds(base, _BPW)])
    pltpu.sync_copy(estage, escore_hbm.at[pl.ds(base, _BPW)])


def _nll_body(d_ref, e_ref, o_ref):
    t = jnp.sum(jnp.log(d_ref[...])) - jnp.sum(jnp.log(e_ref[...]))
    o_ref[0, 0] = t * (1.0 / _BATCH)


def _nll(denom, escore):
    return pl.pallas_call(
        _nll_body,
        out_shape=jax.ShapeDtypeStruct((1, 1), jnp.float32),
        out_specs=pl.BlockSpec(memory_space=pltpu.SMEM),
    )(denom.reshape(8, 128), escore.reshape(8, 128))


def kernel(center_words, target_words, all_vocabs, embedding_v, embedding_u):
    c32 = center_words.reshape(-1).astype(jnp.int32)
    t32 = target_words.reshape(-1).astype(jnp.int32)
    a32 = all_vocabs.astype(jnp.int32).reshape(-1)
    em = _mm_exp(embedding_v, embedding_u)
    denom, escore = _sc_gather(em, c32, t32, a32)
    return _nll(denom, escore)[0, 0]


# speedup vs baseline: 90.8255x; 1.0999x over previous
import functools

import jax
import jax.numpy as jnp
from jax import lax
from jax.experimental import pallas as pl
from jax.experimental.pallas import tpu as pltpu
from jax.experimental.pallas import tpu_sc as plsc

_VOCAB = 1000
_VPAD = 1024
_NSUB = _VPAD // 128
_EMBED = 128
_BATCH = 1024
_NC = 2
_NS = 16
_NW = _NC * _NS
_BPW = _BATCH // _NW
_L = 16
_NFULL = _VOCAB // _L
_TAIL_START = _VOCAB - _L
_TAIL_KEEP = _NFULL * _L - _TAIL_START


def _mm_exp_body(v_ref, u_ref, em_ref):
    m = lax.dot_general(v_ref[...], u_ref[...],
                        dimension_numbers=(((1,), (1,)), ((), ())),
                        preferred_element_type=jnp.float32)
    em = jnp.exp(m)
    for s in range(_NSUB):
        em_ref[pl.ds(s * _VOCAB, _VOCAB), :] = em[:, s * 128:(s + 1) * 128]


def _mm_exp(v, u_pad):
    return pl.pallas_call(
        _mm_exp_body,
        out_shape=jax.ShapeDtypeStruct((_NSUB * _VOCAB, 128), jnp.float32),
    )(v, u_pad)


_sc_mesh = plsc.VectorSubcoreMesh(core_axis_name="c", subcore_axis_name="s")


@functools.partial(
    pl.kernel,
    mesh=_sc_mesh,
    compiler_params=pltpu.CompilerParams(
        use_tc_tiling_on_sc=False, needs_layout_passes=False,
        disable_bounds_checks=True),
    out_type=(
        jax.ShapeDtypeStruct((_BATCH,), jnp.float32),
        jax.ShapeDtypeStruct((_BATCH,), jnp.float32),
    ),
    scratch_types=[
        pltpu.VMEM((_BPW,), jnp.int32),
        pltpu.VMEM((_BPW,), jnp.int32),
        pltpu.VMEM((_BPW * _NSUB // 2,), jnp.int32),
        pltpu.VMEM((_BPW * _NSUB // 2,), jnp.int32),
        pltpu.VMEM((_BPW, _VOCAB), jnp.int32),
        pltpu.VMEM((_BPW * _NSUB // 2, 128), jnp.float32),
        pltpu.VMEM((_BPW * _NSUB // 2, 128), jnp.float32),
        pltpu.VMEM((_BPW,), jnp.float32),
        pltpu.VMEM((_BPW,), jnp.float32),
        pltpu.SemaphoreType.DMA,
        pltpu.SemaphoreType.DMA,
        pltpu.SemaphoreType.DMA,
    ],
)
def _sc_gather(em_hbm, c_hbm, t_hbm, a_hbm, denom_hbm, escore_hbm,
               cidx, tidx, c8a, c8b, av, rows_a, rows_b, dstage, estage,
               sem_a, sem_b, sem_v):
    wid = lax.axis_index("s") * _NC + lax.axis_index("c")
    base = wid * _BPW
    pltpu.sync_copy(c_hbm.at[pl.ds(base, _BPW)], cidx)
    av_cp = pltpu.async_copy(a_hbm.at[pl.ds(base, _BPW)], av, sem_v)

    lanes = lax.iota(jnp.int32, _L)
    smul = (lanes & 7) * _VOCAB
    half = _BPW * _NSUB // 2
    for c in range(half // _L):
        rsel = (lanes >> 3) + 2 * c
        c8a[pl.ds(c * _L, _L)] = smul + plsc.load_gather(cidx, [rsel])
        c8b[pl.ds(c * _L, _L)] = smul + plsc.load_gather(cidx, [rsel + _L])
    cp_a = pltpu.async_copy(em_hbm.at[c8a], rows_a, sem_a)
    cp_b = pltpu.async_copy(em_hbm.at[c8b], rows_b, sem_b)
    pltpu.sync_copy(t_hbm.at[pl.ds(base, _BPW)], tidx)
    av_cp.wait()
    cp_a.wait()
    cp_b.wait()

    def _make_row(rows_ref, g):
        def _row(i, dsums):
            roff = jnp.full((_L,), 0, jnp.int32) + (i - g * _L) * _NSUB
            acc = jnp.zeros((_L,), jnp.float32)
            for j in range(_NFULL):
                col = av[i, pl.ds(j * _L, _L)]
                acc = acc + plsc.load_gather(
                    rows_ref, [roff + (col >> 7), col & 127])
            col = av[i, pl.ds(_TAIL_START, _L)]
            tail = plsc.load_gather(rows_ref, [roff + (col >> 7), col & 127])
            acc = acc + jnp.where(lanes >= _TAIL_KEEP, tail, 0.0)
            s = jnp.sum(acc)
            return jnp.where(lanes == (i % _L), s, dsums)
        return _row

    for g, rows_ref in enumerate((rows_a, rows_b)):
        dsums = lax.fori_loop(g * _L, (g + 1) * _L, _make_row(rows_ref, g),
                              jnp.zeros((_L,), jnp.float32))
        dstage[pl.ds(g * _L, _L)] = dsums
        tcol = tidx[pl.ds(g * _L, _L)]
        estage[pl.ds(g * _L, _L)] = plsc.load_gather(
            rows_ref, [lanes * _NSUB + (tcol >> 7), tcol & 127])

    pltpu.sync_copy(dstage, denom_hbm.at[pl.ds(base, _BPW)])
    pltpu.sync_copy(estage, escore_hbm.at[pl.ds(base, _BPW)])


def _nll_body(d_ref, e_ref, o_ref):
    t = jnp.sum(jnp.log(d_ref[...])) - jnp.sum(jnp.log(e_ref[...]))
    o_ref[0, 0] = t * (1.0 / _BATCH)


def _nll(denom, escore):
    return pl.pallas_call(
        _nll_body,
        out_shape=jax.ShapeDtypeStruct((1, 1), jnp.float32),
        out_specs=pl.BlockSpec(memory_space=pltpu.SMEM),
    )(denom.reshape(8, 128), escore.reshape(8, 128))


def kernel(center_words, target_words, all_vocabs, embedding_v, embedding_u):
    c32 = center_words.reshape(-1).astype(jnp.int32)
    t32 = target_words.reshape(-1).astype(jnp.int32)
    a32 = all_vocabs.astype(jnp.int32)
    u_pad = jnp.pad(embedding_u, ((0, _VPAD - _VOCAB), (0, 0)))
    em = _mm_exp(embedding_v, u_pad)
    denom, escore = _sc_gather(em, c32, t32, a32)
    return _nll(denom, escore)[0, 0]


# speedup vs baseline: 104.5344x; 1.1509x over previous
import functools

import jax
import jax.numpy as jnp
from jax import lax
from jax.experimental import pallas as pl
from jax.experimental.pallas import tpu as pltpu
from jax.experimental.pallas import tpu_sc as plsc

_VOCAB = 1000
_VPAD = 1024
_NSUB = _VPAD // 128
_EMBED = 128
_BATCH = 1024
_NC = 2
_NS = 16
_NW = _NC * _NS
_BPW = _BATCH // _NW
_L = 16
_NFULL = _VOCAB // _L
_TAIL_START = _VOCAB - _L
_TAIL_KEEP = _NFULL * _L - _TAIL_START


def _mm_exp_body(v_ref, u_ref, em_ref):
    m = lax.dot_general(v_ref[...], u_ref[...],
                        dimension_numbers=(((1,), (1,)), ((), ())),
                        preferred_element_type=jnp.float32)
    em = jnp.exp(m)
    for s in range(_NSUB):
        em_ref[pl.ds(s * _VOCAB, _VOCAB), :] = em[:, s * 128:(s + 1) * 128]


def _mm_exp(v, u_pad):
    return pl.pallas_call(
        _mm_exp_body,
        out_shape=jax.ShapeDtypeStruct((_NSUB * _VOCAB, 128), jnp.float32),
    )(v, u_pad)


_sc_mesh = plsc.VectorSubcoreMesh(core_axis_name="c", subcore_axis_name="s")


@functools.partial(
    pl.kernel,
    mesh=_sc_mesh,
    compiler_params=pltpu.CompilerParams(
        use_tc_tiling_on_sc=False, needs_layout_passes=False,
        disable_bounds_checks=True),
    out_type=(
        jax.ShapeDtypeStruct((_BATCH,), jnp.float32),
        jax.ShapeDtypeStruct((_BATCH,), jnp.float32),
    ),
    scratch_types=[
        pltpu.VMEM((_BPW,), jnp.int32),
        pltpu.VMEM((_BPW,), jnp.int32),
        pltpu.VMEM((_BPW * _NSUB // 2,), jnp.int32),
        pltpu.VMEM((_BPW * _NSUB // 2,), jnp.int32),
        pltpu.VMEM((_VOCAB, _BPW), jnp.int32),
        pltpu.VMEM((_BPW * _NSUB // 2, 128), jnp.float32),
        pltpu.VMEM((_BPW * _NSUB // 2, 128), jnp.float32),
        pltpu.VMEM((_BPW,), jnp.float32),
        pltpu.VMEM((_BPW,), jnp.float32),
        pltpu.SemaphoreType.DMA,
        pltpu.SemaphoreType.DMA,
        pltpu.SemaphoreType.DMA,
    ],
)
def _sc_gather(em_hbm, c_hbm, t_hbm, a_hbm, denom_hbm, escore_hbm,
               cidx, tidx, c8a, c8b, av, rows_a, rows_b, dstage, estage,
               sem_a, sem_b, sem_v):
    wid = lax.axis_index("s") * _NC + lax.axis_index("c")
    base = wid * _BPW
    pltpu.sync_copy(c_hbm.at[pl.ds(base, _BPW)], cidx)
    av_cp = pltpu.async_copy(a_hbm.at[:, pl.ds(base, _BPW)], av, sem_v)

    lanes = lax.iota(jnp.int32, _L)
    smul = (lanes & 7) * _VOCAB
    half = _BPW * _NSUB // 2
    for c in range(half // _L):
        rsel = (lanes >> 3) + 2 * c
        c8a[pl.ds(c * _L, _L)] = smul + plsc.load_gather(cidx, [rsel])
        c8b[pl.ds(c * _L, _L)] = smul + plsc.load_gather(cidx, [rsel + _L])
    cp_a = pltpu.async_copy(em_hbm.at[c8a], rows_a, sem_a)
    cp_b = pltpu.async_copy(em_hbm.at[c8b], rows_b, sem_b)
    pltpu.sync_copy(t_hbm.at[pl.ds(base, _BPW)], tidx)
    av_cp.wait()
    cp_a.wait()
    cp_b.wait()

    lanes8 = lanes * _NSUB
    _UNROLL = 8

    def _make_chunk(rows_ref, g):
        def _chunk(jo, acc):
            for jj in range(_UNROLL):
                col = av[jo * _UNROLL + jj, pl.ds(g * _L, _L)]
                acc = acc + plsc.load_gather(
                    rows_ref, [lanes8 + (col >> 7), col & 127])
            return acc
        return _chunk

    for g, rows_ref in enumerate((rows_a, rows_b)):
        dsums = lax.fori_loop(0, _VOCAB // _UNROLL, _make_chunk(rows_ref, g),
                              jnp.zeros((_L,), jnp.float32))
        dstage[pl.ds(g * _L, _L)] = dsums
        tcol = tidx[pl.ds(g * _L, _L)]
        estage[pl.ds(g * _L, _L)] = plsc.load_gather(
            rows_ref, [lanes * _NSUB + (tcol >> 7), tcol & 127])

    pltpu.sync_copy(dstage, denom_hbm.at[pl.ds(base, _BPW)])
    pltpu.sync_copy(estage, escore_hbm.at[pl.ds(base, _BPW)])


def _nll_body(d_ref, e_ref, o_ref):
    t = jnp.sum(jnp.log(d_ref[...])) - jnp.sum(jnp.log(e_ref[...]))
    o_ref[0, 0] = t * (1.0 / _BATCH)


def _nll(denom, escore):
    return pl.pallas_call(
        _nll_body,
        out_shape=jax.ShapeDtypeStruct((1, 1), jnp.float32),
        out_specs=pl.BlockSpec(memory_space=pltpu.SMEM),
    )(denom.reshape(8, 128), escore.reshape(8, 128))


def kernel(center_words, target_words, all_vocabs, embedding_v, embedding_u):
    c32 = center_words.reshape(-1).astype(jnp.int32)
    t32 = target_words.reshape(-1).astype(jnp.int32)
    a32 = all_vocabs.astype(jnp.int32).T
    u_pad = jnp.pad(embedding_u, ((0, _VPAD - _VOCAB), (0, 0)))
    em = _mm_exp(embedding_v, u_pad)
    denom, escore = _sc_gather(em, c32, t32, a32)
    return _nll(denom, escore)[0, 0]


# speedup vs baseline: 104.7069x; 1.0016x over previous
import functools

import jax
import jax.numpy as jnp
from jax import lax
from jax.experimental import pallas as pl
from jax.experimental.pallas import tpu as pltpu
from jax.experimental.pallas import tpu_sc as plsc

_VOCAB = 1000
_VPAD = 1024
_NSUB = _VPAD // 128
_EMBED = 128
_BATCH = 1024
_NC = 2
_NS = 16
_NW = _NC * _NS
_BPW = _BATCH // _NW
_L = 16
_NFULL = _VOCAB // _L
_TAIL_START = _VOCAB - _L
_TAIL_KEEP = _NFULL * _L - _TAIL_START


def _mm_exp_body(v_ref, u_ref, em_ref):
    m = lax.dot_general(v_ref[...], u_ref[...],
                        dimension_numbers=(((1,), (1,)), ((), ())),
                        preferred_element_type=jnp.float32)
    em = jnp.exp(m)
    for s in range(_NSUB):
        em_ref[pl.ds(s * _VOCAB, _VOCAB), :] = em[:, s * 128:(s + 1) * 128]


def _mm_exp(v, u_pad):
    return pl.pallas_call(
        _mm_exp_body,
        out_shape=jax.ShapeDtypeStruct((_NSUB * _VOCAB, 128), jnp.float32),
    )(v, u_pad)


_sc_mesh = plsc.VectorSubcoreMesh(core_axis_name="c", subcore_axis_name="s")


@functools.partial(
    pl.kernel,
    mesh=_sc_mesh,
    compiler_params=pltpu.CompilerParams(
        use_tc_tiling_on_sc=False, needs_layout_passes=False,
        disable_bounds_checks=True),
    out_type=(
        jax.ShapeDtypeStruct((_BATCH,), jnp.float32),
        jax.ShapeDtypeStruct((_BATCH,), jnp.float32),
    ),
    scratch_types=[
        pltpu.VMEM((_BPW,), jnp.int32),
        pltpu.VMEM((_BPW,), jnp.int32),
        pltpu.VMEM((_BPW * _NSUB // 2,), jnp.int32),
        pltpu.VMEM((_BPW * _NSUB // 2,), jnp.int32),
        pltpu.VMEM((_VOCAB, _BPW), jnp.int32),
        pltpu.VMEM((_BPW * _NSUB // 2, 128), jnp.float32),
        pltpu.VMEM((_BPW * _NSUB // 2, 128), jnp.float32),
        pltpu.VMEM((_BPW,), jnp.float32),
        pltpu.VMEM((_BPW,), jnp.float32),
        pltpu.SemaphoreType.DMA,
        pltpu.SemaphoreType.DMA,
        pltpu.SemaphoreType.DMA,
    ],
)
def _sc_gather(em_hbm, c_hbm, t_hbm, a_hbm, denom_hbm, escore_hbm,
               cidx, tidx, c8a, c8b, av, rows_a, rows_b, dstage, estage,
               sem_a, sem_b, sem_v):
    wid = lax.axis_index("s") * _NC + lax.axis_index("c")
    base = wid * _BPW
    pltpu.sync_copy(c_hbm.at[pl.ds(base, _BPW)], cidx)
    av_cp = pltpu.async_copy(a_hbm.at[:, pl.ds(base, _BPW)], av, sem_v)

    lanes = lax.iota(jnp.int32, _L)
    smul = (lanes & 7) * _VOCAB
    half = _BPW * _NSUB // 2
    for c in range(half // _L):
        rsel = (lanes >> 3) + 2 * c
        c8a[pl.ds(c * _L, _L)] = smul + plsc.load_gather(cidx, [rsel])
        c8b[pl.ds(c * _L, _L)] = smul + plsc.load_gather(cidx, [rsel + _L])
    cp_a = pltpu.async_copy(em_hbm.at[c8a], rows_a, sem_a)
    cp_b = pltpu.async_copy(em_hbm.at[c8b], rows_b, sem_b)
    pltpu.sync_copy(t_hbm.at[pl.ds(base, _BPW)], tidx)
    av_cp.wait()
    cp_a.wait()
    cp_b.wait()

    lanes8 = lanes * _NSUB
    _UNROLL = 4

    def _make_chunk(rows_ref, g):
        def _chunk(jo, acc):
            for jj in range(_UNROLL):
                col = av[jo * _UNROLL + jj, pl.ds(g * _L, _L)]
                acc = acc + plsc.load_gather(
                    rows_ref, [lanes8 + (col >> 7), col & 127])
            return acc
        return _chunk

    for g, rows_ref in enumerate((rows_a, rows_b)):
        dsums = lax.fori_loop(0, _VOCAB // _UNROLL, _make_chunk(rows_ref, g),
                              jnp.zeros((_L,), jnp.float32))
        dstage[pl.ds(g * _L, _L)] = dsums
        tcol = tidx[pl.ds(g * _L, _L)]
        estage[pl.ds(g * _L, _L)] = plsc.load_gather(
            rows_ref, [lanes * _NSUB + (tcol >> 7), tcol & 127])

    pltpu.sync_copy(dstage, denom_hbm.at[pl.ds(base, _BPW)])
    pltpu.sync_copy(estage, escore_hbm.at[pl.ds(base, _BPW)])


def _nll_body(d_ref, e_ref, o_ref):
    t = jnp.sum(jnp.log(d_ref[...])) - jnp.sum(jnp.log(e_ref[...]))
    o_ref[0, 0] = t * (1.0 / _BATCH)


def _nll(denom, escore):
    return pl.pallas_call(
        _nll_body,
        out_shape=jax.ShapeDtypeStruct((1, 1), jnp.float32),
        out_specs=pl.BlockSpec(memory_space=pltpu.SMEM),
    )(denom.reshape(8, 128), escore.reshape(8, 128))


def kernel(center_words, target_words, all_vocabs, embedding_v, embedding_u):
    c32 = center_words.reshape(-1).astype(jnp.int32)
    t32 = target_words.reshape(-1).astype(jnp.int32)
    a32 = all_vocabs.astype(jnp.int32).T
    u_pad = jnp.pad(embedding_u, ((0, _VPAD - _VOCAB), (0, 0)))
    em = _mm_exp(embedding_v, u_pad)
    denom, escore = _sc_gather(em, c32, t32, a32)
    return _nll(denom, escore)[0, 0]
